# baseline (speedup 1.0000x reference)
import functools

import jax
import jax.numpy as jnp
from jax import lax
from jax.experimental import pallas as pl
from jax.experimental.pallas import tpu as pltpu
from jax.experimental.pallas import tpu_sc as plsc

N = 10000
NPAD = 10240
E = 320000
HEADS = 8

NC = 2
NS = 16
NW = NC * NS
EPW = E // NW
K = 40
NCHUNK = EPW // K
NB = 5
NSTEP = NCHUNK // NB
RPW = NPAD // NS

_TCB = 1000


def _leaky(v):
  return jnp.maximum(v, v * jnp.float32(0.2))


def _prep_body(x_ref, w_ref, as_ref, ad_ref, hx_ref, d_ref):
  h = jnp.dot(x_ref[...], w_ref[...], preferred_element_type=jnp.float32)
  dout = w_ref.shape[1]
  hx_ref[:, :dout] = h
  hx_ref[:, dout:] = jnp.dot(h, as_ref[...],
                             preferred_element_type=jnp.float32)
  d_ref[...] = jnp.dot(h, ad_ref[...], preferred_element_type=jnp.float32)


def _tc_prep(x, W, As, Ad):
  dout = W.shape[1]
  grid = N // _TCB
  return pl.pallas_call(
      _prep_body,
      grid=(grid,),
      in_specs=[
          pl.BlockSpec((_TCB, x.shape[1]), lambda i: (i, 0)),
          pl.BlockSpec(W.shape, lambda i: (0, 0)),
          pl.BlockSpec(As.shape, lambda i: (0, 0)),
          pl.BlockSpec(Ad.shape, lambda i: (0, 0)),
      ],
      out_specs=[
          pl.BlockSpec((_TCB, dout + 16), lambda i: (i, 0)),
          pl.BlockSpec((_TCB, 16), lambda i: (i, 0)),
      ],
      out_shape=[
          jax.ShapeDtypeStruct((N, dout + 16), jnp.float32),
          jax.ShapeDtypeStruct((N, 16), jnp.float32),
      ],
  )(x, W, As, Ad)


def _fin_prep_body(p_ref, hx_ref, dd_ref, xres_ref, b_ref, e16_ref,
                   w_ref, as_ref, ad_ref,
                   xn_ref, hxn_ref, dn_ref):
  hx = hx_ref[...]
  h = hx[:, :128]
  s = hx[:, 128:144]
  exl = jnp.exp(_leaky(s + dd_ref[...]))
  e16 = e16_ref[...]
  exlb = jnp.dot(exl, e16, preferred_element_type=jnp.float32)
  psum = p_ref[0] + p_ref[1]
  num = psum[:, :128] + exlb * h
  den = jnp.dot(psum[:, 128:144] + exl, e16,
                preferred_element_type=jnp.float32) + jnp.float32(1e-16)
  agg = num / den + b_ref[...]
  xn = jnp.where(agg > 0, agg, jnp.exp(agg) - jnp.float32(1.0)) + xres_ref[...]
  xn_ref[...] = xn
  hn = jnp.dot(xn, w_ref[...], preferred_element_type=jnp.float32)
  dnext = w_ref.shape[1]
  hxn_ref[:, :dnext] = hn
  hxn_ref[:, dnext:] = jnp.dot(hn, as_ref[...],
                               preferred_element_type=jnp.float32)
  dn_ref[...] = jnp.dot(hn, ad_ref[...], preferred_element_type=jnp.float32)


def _tc_fin_prep(P, hx, D, xres, b, e16, W, As, Ad):
  dnext = W.shape[1]
  grid = N // _TCB
  blk = lambda shp: pl.BlockSpec(shp, lambda i: tuple(0 for _ in shp))
  row = lambda w: pl.BlockSpec((_TCB, w), lambda i: (i, 0))
  prow = lambda w: pl.BlockSpec((2, _TCB, w), lambda i: (0, i, 0))
  return pl.pallas_call(
      _fin_prep_body,
      grid=(grid,),
      in_specs=[prow(144), row(144), row(16), row(128), blk((1, 128)),
                blk((16, 128)), blk(W.shape), blk(As.shape), blk(Ad.shape)],
      out_specs=[row(128), row(dnext + 16), row(16)],
      out_shape=[
          jax.ShapeDtypeStruct((N, 128), jnp.float32),
          jax.ShapeDtypeStruct((N, dnext + 16), jnp.float32),
          jax.ShapeDtypeStruct((N, 16), jnp.float32),
      ],
  )(P, hx, D, xres, b, e16, W, As, Ad)


def _final_body(p_ref, hx_ref, dd_ref, b_ref, out_ref):
  hx = hx_ref[...]
  h2 = hx[:, :64]
  s2 = hx[:, 64:80]
  exl = jnp.exp(_leaky(s2 + dd_ref[...]))
  exl0 = exl[:, 0:1]
  psum = p_ref[0] + p_ref[1]
  den = psum[:, 64:65] + exl0 + jnp.float32(1e-16)
  o = (psum[:, :64] + exl0 * h2) / den + b_ref[...]
  m = jnp.max(o, axis=1, keepdims=True)
  om = o - m
  out_ref[...] = om - jnp.log(jnp.sum(jnp.exp(om), axis=1, keepdims=True))


def _tc_final(P, hx, D, b2):
  grid = N // _TCB
  blk = lambda shp: pl.BlockSpec(shp, lambda i: tuple(0 for _ in shp))
  row = lambda w: pl.BlockSpec((_TCB, w), lambda i: (i, 0))
  prow = lambda w: pl.BlockSpec((2, _TCB, w), lambda i: (0, i, 0))
  return pl.pallas_call(
      _final_body,
      grid=(grid,),
      in_specs=[prow(80), row(80), row(16), blk((1, 64))],
      out_specs=row(64),
      out_shape=jax.ShapeDtypeStruct((N, 64), jnp.float32),
  )(P, hx, D, b2)


def _make_sc_edge_pass(d_feat, heads):
  dext = d_feat + 16
  n_grp = d_feat // 16
  rep = n_grp // heads
  mesh = plsc.VectorSubcoreMesh(core_axis_name="c", subcore_axis_name="s",
                                num_cores=NC, num_subcores=NS)

  @functools.partial(
      pl.kernel,
      out_type=jax.ShapeDtypeStruct((NC, NPAD, dext), jnp.float32),
      mesh=mesh,
      compiler_params=pltpu.CompilerParams(use_tc_tiling_on_sc=False),
      scratch_types=[
          pltpu.VMEM((NB, K), jnp.int32),
          pltpu.VMEM((NB, K), jnp.int32),
          pltpu.VMEM((NB, K, 16), jnp.float32),
          pltpu.VMEM((NB, K, dext), jnp.float32),
          pltpu.VMEM_SHARED((NPAD, dext), jnp.float32),
          pltpu.SemaphoreType.DMA((NB,)),
          pltpu.SemaphoreType.DMA((NB,)),
          pltpu.SemaphoreType.DMA((NB,)),
      ],
  )
  def sc_edge(hx_hbm, d_hbm, src_hbm, dst_hbm, zrow_hbm, part_hbm,
              srcv, dstv, dr, hr, acc, sidx, sg, ssc):
    c = lax.axis_index("c")
    s = lax.axis_index("s")
    wid = c * NS + s
    row0 = s * RPW

    pltpu.sync_copy(zrow_hbm, acc.at[pl.ds(row0, RPW)])
    plsc.subcore_barrier()

    base = wid * EPW

    def step_body(st, carry):
      c0 = st * NB
      d_idx = []
      for b in range(NB):
        off = base + (c0 + b) * K
        d_idx.append((
            pltpu.async_copy(src_hbm.at[pl.ds(off, K)], srcv.at[b],
                             sidx.at[b]),
            pltpu.async_copy(dst_hbm.at[pl.ds(off, K)], dstv.at[b],
                             sidx.at[b])))
      d_g = []
      for b in range(NB):
        for d in d_idx[b]:
          d.wait()
        g1 = pltpu.async_copy(hx_hbm.at[srcv.at[b]], hr.at[b], sg.at[b])
        g2 = pltpu.async_copy(d_hbm.at[dstv.at[b]], dr.at[b], sg.at[b])
        d_g.append((g1, g2))
      d_sc = []
      for b in range(NB):
        for d in d_g[b]:
          d.wait()

        def edge_body(e, _b=b):
          sv = hr[_b, e, pl.ds(d_feat, 16)]
          ex = jnp.exp(_leaky(sv + dr[_b, e]))
          hr[_b, e, pl.ds(d_feat, 16)] = ex
          for g in range(heads):
            w = ex[jnp.full((16,), g, jnp.int32)]
            for q in range(rep):
              col = (g * rep + q) * 16
              hr[_b, e, pl.ds(col, 16)] = hr[_b, e, pl.ds(col, 16)] * w

        plsc.parallel_loop(0, K, unroll=6)(edge_body)
        d_sc.append(pltpu.async_copy(hr.at[b], acc.at[dstv.at[b]], ssc.at[b],
                                     add=True))
      for d in d_sc:
        d.wait()
      return carry

    lax.fori_loop(0, NSTEP, step_body, 0)
    plsc.subcore_barrier()

    pltpu.sync_copy(acc.at[pl.ds(row0, RPW)], part_hbm.at[c, pl.ds(row0, RPW)])

  return sc_edge


_sc_edge_128 = _make_sc_edge_pass(128, HEADS)
_sc_edge_64 = _make_sc_edge_pass(64, 1)


def _attn_block(a_src, a_dst, d_feat):
  heads, dh = a_src.shape
  eye = jnp.eye(heads, dtype=jnp.float32)
  def mk(a):
    blk = a[:, :, None] * eye[:, None, :]
    m = blk.reshape(d_feat, heads)
    return jnp.pad(m, ((0, 0), (0, 16 - heads)))
  return mk(a_src), mk(a_dst)


def kernel(x, edge_index, W0, a_src0, a_dst0, b0, W1, a_src1, a_dst1, b1,
           W2, a_src2, a_dst2, b2):
  x = x.astype(jnp.float32)
  src = edge_index[0].astype(jnp.int32)
  dst = edge_index[1].astype(jnp.int32)

  As0, Ad0 = _attn_block(a_src0, a_dst0, 128)
  As1, Ad1 = _attn_block(a_src1, a_dst1, 128)
  As2, Ad2 = _attn_block(a_src2, a_dst2, 64)

  e16 = (jnp.eye(8, dtype=jnp.float32)[:, :, None]
         * jnp.ones((16,), jnp.float32)).reshape(8, 128)
  e16 = jnp.pad(e16, ((0, 8), (0, 0)))

  z144 = jnp.zeros((RPW, 144), jnp.float32)
  z80 = jnp.zeros((RPW, 80), jnp.float32)

  h0x, D0 = _tc_prep(x, W0, As0, Ad0)
  p = _sc_edge_128(h0x, D0, src, dst, z144)
  x1, h1x, D1 = _tc_fin_prep(p, h0x, D0, x, b0.reshape(1, 128), e16,
                             W1, As1, Ad1)
  p = _sc_edge_128(h1x, D1, src, dst, z144)
  x2, h2x, D2 = _tc_fin_prep(p, h1x, D1, x1, b1.reshape(1, 128), e16,
                             W2, As2, Ad2)
  p = _sc_edge_64(h2x, D2, src, dst, z80)
  out = _tc_final(p, h2x, D2, b2.reshape(1, 64))
  return out


# speedup vs baseline: 1.1324x; 1.1324x over previous
import functools

import jax
import jax.numpy as jnp
from jax import lax
from jax.experimental import pallas as pl
from jax.experimental.pallas import tpu as pltpu
from jax.experimental.pallas import tpu_sc as plsc

N = 10000
NPAD = 10240
E = 320000
HEADS = 8

NC = 2
NS = 16
NW = NC * NS
EPW = E // NW
K = 40
NCHUNK = EPW // K
NB = 5
NSTEP = NCHUNK // NB
RPW = NPAD // NS

_TCB = 1000


def _leaky(v):
  return jnp.maximum(v, v * jnp.float32(0.2))


def _prep_body(x_ref, w_ref, as_ref, ad_ref, hx_ref, d_ref):
  h = jnp.dot(x_ref[...], w_ref[...], preferred_element_type=jnp.float32)
  dout = w_ref.shape[1]
  hx_ref[:, :dout] = h
  hx_ref[:, dout:] = jnp.dot(h, as_ref[...],
                             preferred_element_type=jnp.float32)
  d_ref[...] = jnp.dot(h, ad_ref[...], preferred_element_type=jnp.float32)


def _tc_prep(x, W, As, Ad):
  dout = W.shape[1]
  grid = N // _TCB
  return pl.pallas_call(
      _prep_body,
      grid=(grid,),
      in_specs=[
          pl.BlockSpec((_TCB, x.shape[1]), lambda i: (i, 0)),
          pl.BlockSpec(W.shape, lambda i: (0, 0)),
          pl.BlockSpec(As.shape, lambda i: (0, 0)),
          pl.BlockSpec(Ad.shape, lambda i: (0, 0)),
      ],
      out_specs=[
          pl.BlockSpec((_TCB, dout + 16), lambda i: (i, 0)),
          pl.BlockSpec((_TCB, 16), lambda i: (i, 0)),
      ],
      out_shape=[
          jax.ShapeDtypeStruct((N, dout + 16), jnp.float32),
          jax.ShapeDtypeStruct((N, 16), jnp.float32),
      ],
  )(x, W, As, Ad)


def _fin_prep_body(p_ref, hx_ref, dd_ref, xres_ref, b_ref, e16_ref,
                   w_ref, as_ref, ad_ref,
                   xn_ref, hxn_ref, dn_ref):
  hx = hx_ref[...]
  h = hx[:, :128]
  s = hx[:, 128:144]
  exl = jnp.exp(_leaky(s + dd_ref[...]))
  e16 = e16_ref[...]
  exlb = jnp.dot(exl, e16, preferred_element_type=jnp.float32)
  psum = p_ref[0] + p_ref[1]
  num = psum[:, :128] + exlb * h
  den = jnp.dot(psum[:, 128:144] + exl, e16,
                preferred_element_type=jnp.float32) + jnp.float32(1e-16)
  agg = num / den + b_ref[...]
  xn = jnp.where(agg > 0, agg, jnp.exp(agg) - jnp.float32(1.0)) + xres_ref[...]
  xn_ref[...] = xn
  hn = jnp.dot(xn, w_ref[...], preferred_element_type=jnp.float32)
  dnext = w_ref.shape[1]
  hxn_ref[:, :dnext] = hn
  hxn_ref[:, dnext:] = jnp.dot(hn, as_ref[...],
                               preferred_element_type=jnp.float32)
  dn_ref[...] = jnp.dot(hn, ad_ref[...], preferred_element_type=jnp.float32)


def _tc_fin_prep(P, hx, D, xres, b, e16, W, As, Ad):
  dnext = W.shape[1]
  grid = N // _TCB
  blk = lambda shp: pl.BlockSpec(shp, lambda i: tuple(0 for _ in shp))
  row = lambda w: pl.BlockSpec((_TCB, w), lambda i: (i, 0))
  prow = lambda w: pl.BlockSpec((2, _TCB, w), lambda i: (0, i, 0))
  return pl.pallas_call(
      _fin_prep_body,
      grid=(grid,),
      in_specs=[prow(144), row(144), row(16), row(128), blk((1, 128)),
                blk((16, 128)), blk(W.shape), blk(As.shape), blk(Ad.shape)],
      out_specs=[row(128), row(dnext + 16), row(16)],
      out_shape=[
          jax.ShapeDtypeStruct((N, 128), jnp.float32),
          jax.ShapeDtypeStruct((N, dnext + 16), jnp.float32),
          jax.ShapeDtypeStruct((N, 16), jnp.float32),
      ],
  )(P, hx, D, xres, b, e16, W, As, Ad)


def _final_body(p_ref, hx_ref, dd_ref, b_ref, out_ref):
  hx = hx_ref[...]
  h2 = hx[:, :64]
  s2 = hx[:, 64:80]
  exl = jnp.exp(_leaky(s2 + dd_ref[...]))
  exl0 = exl[:, 0:1]
  psum = p_ref[0] + p_ref[1]
  den = psum[:, 64:65] + exl0 + jnp.float32(1e-16)
  o = (psum[:, :64] + exl0 * h2) / den + b_ref[...]
  m = jnp.max(o, axis=1, keepdims=True)
  om = o - m
  out_ref[...] = om - jnp.log(jnp.sum(jnp.exp(om), axis=1, keepdims=True))


def _tc_final(P, hx, D, b2):
  grid = N // _TCB
  blk = lambda shp: pl.BlockSpec(shp, lambda i: tuple(0 for _ in shp))
  row = lambda w: pl.BlockSpec((_TCB, w), lambda i: (i, 0))
  prow = lambda w: pl.BlockSpec((2, _TCB, w), lambda i: (0, i, 0))
  return pl.pallas_call(
      _final_body,
      grid=(grid,),
      in_specs=[prow(80), row(80), row(16), blk((1, 64))],
      out_specs=row(64),
      out_shape=jax.ShapeDtypeStruct((N, 64), jnp.float32),
  )(P, hx, D, b2)


def _make_sc_edge_pass(d_feat, heads):
  dext = d_feat + 16
  n_grp = d_feat // 16
  rep = n_grp // heads
  mesh = plsc.VectorSubcoreMesh(core_axis_name="c", subcore_axis_name="s",
                                num_cores=NC, num_subcores=NS)

  @functools.partial(
      pl.kernel,
      out_type=jax.ShapeDtypeStruct((NC, NPAD, dext), jnp.float32),
      mesh=mesh,
      compiler_params=pltpu.CompilerParams(use_tc_tiling_on_sc=False),
      scratch_types=[
          pltpu.VMEM((NB, K), jnp.int32),
          pltpu.VMEM((NB, K), jnp.int32),
          pltpu.VMEM((NB, K, 16), jnp.float32),
          pltpu.VMEM((NB, K, dext), jnp.float32),
          pltpu.VMEM_SHARED((NPAD, dext), jnp.float32),
          pltpu.SemaphoreType.DMA((NB,)),
          pltpu.SemaphoreType.DMA((NB,)),
          pltpu.SemaphoreType.DMA((NB,)),
      ],
  )
  def sc_edge(hx_hbm, d_hbm, src_hbm, dst_hbm, zrow_hbm, part_hbm,
              srcv, dstv, dr, hr, acc, sidx, sg, ssc):
    c = lax.axis_index("c")
    s = lax.axis_index("s")
    wid = c * NS + s
    row0 = s * RPW

    pltpu.sync_copy(zrow_hbm, acc.at[pl.ds(row0, RPW)])
    plsc.subcore_barrier()

    base = wid * EPW

    def step_body(st, carry):
      c0 = st * NB
      d_idx = []
      for b in range(NB):
        off = base + (c0 + b) * K
        d_idx.append((
            pltpu.async_copy(src_hbm.at[pl.ds(off, K)], srcv.at[b],
                             sidx.at[b]),
            pltpu.async_copy(dst_hbm.at[pl.ds(off, K)], dstv.at[b],
                             sidx.at[b])))
      d_g = []
      for b in range(NB):
        for d in d_idx[b]:
          d.wait()
        g1 = pltpu.async_copy(hx_hbm.at[srcv.at[b]], hr.at[b], sg.at[b])
        g2 = pltpu.async_copy(d_hbm.at[dstv.at[b]], dr.at[b], sg.at[b])
        d_g.append((g1, g2))
      d_sc = []
      for b in range(NB):
        for d in d_g[b]:
          d.wait()

        def edge_body(e, _b=b):
          sv = hr[_b, e, pl.ds(d_feat, 16)]
          ex = jnp.exp(_leaky(sv + dr[_b, e]))
          hr[_b, e, pl.ds(d_feat, 16)] = ex
          for g in range(heads):
            w = ex[jnp.full((16,), g, jnp.int32)]
            for q in range(rep):
              col = (g * rep + q) * 16
              hr[_b, e, pl.ds(col, 16)] = hr[_b, e, pl.ds(col, 16)] * w

        plsc.parallel_loop(0, K, unroll=2)(edge_body)
        d_sc.append(pltpu.async_copy(hr.at[b], acc.at[dstv.at[b]], ssc.at[b],
                                     add=True))
      for d in d_sc:
        d.wait()
      return carry

    lax.fori_loop(0, NSTEP, step_body, 0)
    plsc.subcore_barrier()

    pltpu.sync_copy(acc.at[pl.ds(row0, RPW)], part_hbm.at[c, pl.ds(row0, RPW)])

  return sc_edge


_sc_edge_128 = _make_sc_edge_pass(128, HEADS)
_sc_edge_64 = _make_sc_edge_pass(64, 1)


def _attn_block(a_src, a_dst, d_feat):
  heads, dh = a_src.shape
  eye = jnp.eye(heads, dtype=jnp.float32)
  def mk(a):
    blk = a[:, :, None] * eye[:, None, :]
    m = blk.reshape(d_feat, heads)
    return jnp.pad(m, ((0, 0), (0, 16 - heads)))
  return mk(a_src), mk(a_dst)


def kernel(x, edge_index, W0, a_src0, a_dst0, b0, W1, a_src1, a_dst1, b1,
           W2, a_src2, a_dst2, b2):
  x = x.astype(jnp.float32)
  src = edge_index[0].astype(jnp.int32)
  dst = edge_index[1].astype(jnp.int32)

  As0, Ad0 = _attn_block(a_src0, a_dst0, 128)
  As1, Ad1 = _attn_block(a_src1, a_dst1, 128)
  As2, Ad2 = _attn_block(a_src2, a_dst2, 64)

  e16 = (jnp.eye(8, dtype=jnp.float32)[:, :, None]
         * jnp.ones((16,), jnp.float32)).reshape(8, 128)
  e16 = jnp.pad(e16, ((0, 8), (0, 0)))

  z144 = jnp.zeros((RPW, 144), jnp.float32)
  z80 = jnp.zeros((RPW, 80), jnp.float32)

  h0x, D0 = _tc_prep(x, W0, As0, Ad0)
  p = _sc_edge_128(h0x, D0, src, dst, z144)
  x1, h1x, D1 = _tc_fin_prep(p, h0x, D0, x, b0.reshape(1, 128), e16,
                             W1, As1, Ad1)
  p = _sc_edge_128(h1x, D1, src, dst, z144)
  x2, h2x, D2 = _tc_fin_prep(p, h1x, D1, x1, b1.reshape(1, 128), e16,
                             W2, As2, Ad2)
  p = _sc_edge_64(h2x, D2, src, dst, z80)
  out = _tc_final(p, h2x, D2, b2.reshape(1, 64))
  return out


# speedup vs baseline: 1.1469x; 1.0128x over previous
import functools

import jax
import jax.numpy as jnp
from jax import lax
from jax.experimental import pallas as pl
from jax.experimental.pallas import tpu as pltpu
from jax.experimental.pallas import tpu_sc as plsc

N = 10000
NPAD = 10240
E = 320000
HEADS = 8

NC = 2
NS = 16
NW = NC * NS
EPW = E // NW
K = 40
NCHUNK = EPW // K
NB = 5
NSTEP = NCHUNK // NB
RPW = NPAD // NS

_TCB = 2000


def _leaky(v):
  return jnp.maximum(v, v * jnp.float32(0.2))


def _prep_body(x_ref, w_ref, as_ref, ad_ref, hx_ref, d_ref):
  h = jnp.dot(x_ref[...], w_ref[...], preferred_element_type=jnp.float32)
  dout = w_ref.shape[1]
  hx_ref[:, :dout] = h
  hx_ref[:, dout:] = jnp.dot(h, as_ref[...],
                             preferred_element_type=jnp.float32)
  d_ref[...] = jnp.dot(h, ad_ref[...], preferred_element_type=jnp.float32)


def _tc_prep(x, W, As, Ad):
  dout = W.shape[1]
  grid = N // _TCB
  return pl.pallas_call(
      _prep_body,
      grid=(grid,),
      in_specs=[
          pl.BlockSpec((_TCB, x.shape[1]), lambda i: (i, 0)),
          pl.BlockSpec(W.shape, lambda i: (0, 0)),
          pl.BlockSpec(As.shape, lambda i: (0, 0)),
          pl.BlockSpec(Ad.shape, lambda i: (0, 0)),
      ],
      out_specs=[
          pl.BlockSpec((_TCB, dout + 16), lambda i: (i, 0)),
          pl.BlockSpec((_TCB, 16), lambda i: (i, 0)),
      ],
      out_shape=[
          jax.ShapeDtypeStruct((N, dout + 16), jnp.float32),
          jax.ShapeDtypeStruct((N, 16), jnp.float32),
      ],
  )(x, W, As, Ad)


def _fin_prep_body(p_ref, hx_ref, dd_ref, xres_ref, b_ref, e16_ref,
                   w_ref, as_ref, ad_ref,
                   xn_ref, hxn_ref, dn_ref):
  hx = hx_ref[...]
  h = hx[:, :128]
  s = hx[:, 128:144]
  exl = jnp.exp(_leaky(s + dd_ref[...]))
  e16 = e16_ref[...]
  exlb = jnp.dot(exl, e16, preferred_element_type=jnp.float32)
  psum = p_ref[0] + p_ref[1]
  num = psum[:, :128] + exlb * h
  den = jnp.dot(psum[:, 128:144] + exl, e16,
                preferred_element_type=jnp.float32) + jnp.float32(1e-16)
  agg = num / den + b_ref[...]
  xn = jnp.where(agg > 0, agg, jnp.exp(agg) - jnp.float32(1.0)) + xres_ref[...]
  xn_ref[...] = xn
  hn = jnp.dot(xn, w_ref[...], preferred_element_type=jnp.float32)
  dnext = w_ref.shape[1]
  hxn_ref[:, :dnext] = hn
  hxn_ref[:, dnext:] = jnp.dot(hn, as_ref[...],
                               preferred_element_type=jnp.float32)
  dn_ref[...] = jnp.dot(hn, ad_ref[...], preferred_element_type=jnp.float32)


def _tc_fin_prep(P, hx, D, xres, b, e16, W, As, Ad):
  dnext = W.shape[1]
  grid = N // _TCB
  blk = lambda shp: pl.BlockSpec(shp, lambda i: tuple(0 for _ in shp))
  row = lambda w: pl.BlockSpec((_TCB, w), lambda i: (i, 0))
  prow = lambda w: pl.BlockSpec((2, _TCB, w), lambda i: (0, i, 0))
  return pl.pallas_call(
      _fin_prep_body,
      grid=(grid,),
      in_specs=[prow(144), row(144), row(16), row(128), blk((1, 128)),
                blk((16, 128)), blk(W.shape), blk(As.shape), blk(Ad.shape)],
      out_specs=[row(128), row(dnext + 16), row(16)],
      out_shape=[
          jax.ShapeDtypeStruct((N, 128), jnp.float32),
          jax.ShapeDtypeStruct((N, dnext + 16), jnp.float32),
          jax.ShapeDtypeStruct((N, 16), jnp.float32),
      ],
  )(P, hx, D, xres, b, e16, W, As, Ad)


def _final_body(p_ref, hx_ref, dd_ref, b_ref, out_ref):
  hx = hx_ref[...]
  h2 = hx[:, :64]
  s2 = hx[:, 64:80]
  exl = jnp.exp(_leaky(s2 + dd_ref[...]))
  exl0 = exl[:, 0:1]
  psum = p_ref[0] + p_ref[1]
  den = psum[:, 64:65] + exl0 + jnp.float32(1e-16)
  o = (psum[:, :64] + exl0 * h2) / den + b_ref[...]
  m = jnp.max(o, axis=1, keepdims=True)
  om = o - m
  out_ref[...] = om - jnp.log(jnp.sum(jnp.exp(om), axis=1, keepdims=True))


def _tc_final(P, hx, D, b2):
  grid = N // _TCB
  blk = lambda shp: pl.BlockSpec(shp, lambda i: tuple(0 for _ in shp))
  row = lambda w: pl.BlockSpec((_TCB, w), lambda i: (i, 0))
  prow = lambda w: pl.BlockSpec((2, _TCB, w), lambda i: (0, i, 0))
  return pl.pallas_call(
      _final_body,
      grid=(grid,),
      in_specs=[prow(80), row(80), row(16), blk((1, 64))],
      out_specs=row(64),
      out_shape=jax.ShapeDtypeStruct((N, 64), jnp.float32),
  )(P, hx, D, b2)


def _make_sc_edge_pass(d_feat, heads):
  dext = d_feat + 16
  n_grp = d_feat // 16
  rep = n_grp // heads
  mesh = plsc.VectorSubcoreMesh(core_axis_name="c", subcore_axis_name="s",
                                num_cores=NC, num_subcores=NS)

  @functools.partial(
      pl.kernel,
      out_type=jax.ShapeDtypeStruct((NC, NPAD, dext), jnp.float32),
      mesh=mesh,
      compiler_params=pltpu.CompilerParams(use_tc_tiling_on_sc=False),
      scratch_types=[
          pltpu.VMEM((NB, K), jnp.int32),
          pltpu.VMEM((NB, K), jnp.int32),
          pltpu.VMEM((NB, K, 16), jnp.float32),
          pltpu.VMEM((NB, K, dext), jnp.float32),
          pltpu.VMEM_SHARED((NPAD, dext), jnp.float32),
          pltpu.SemaphoreType.DMA((NB,)),
          pltpu.SemaphoreType.DMA((NB,)),
          pltpu.SemaphoreType.DMA((NB,)),
      ],
  )
  def sc_edge(hx_hbm, d_hbm, src_hbm, dst_hbm, zrow_hbm, part_hbm,
              srcv, dstv, dr, hr, acc, sidx, sg, ssc):
    c = lax.axis_index("c")
    s = lax.axis_index("s")
    wid = c * NS + s
    row0 = s * RPW

    pltpu.sync_copy(zrow_hbm, acc.at[pl.ds(row0, RPW)])
    plsc.subcore_barrier()

    base = wid * EPW

    def step_body(st, carry):
      c0 = st * NB
      d_idx = []
      for b in range(NB):
        off = base + (c0 + b) * K
        d_idx.append((
            pltpu.async_copy(src_hbm.at[pl.ds(off, K)], srcv.at[b],
                             sidx.at[b]),
            pltpu.async_copy(dst_hbm.at[pl.ds(off, K)], dstv.at[b],
                             sidx.at[b])))
      d_g = []
      for b in range(NB):
        for d in d_idx[b]:
          d.wait()
        g1 = pltpu.async_copy(hx_hbm.at[srcv.at[b]], hr.at[b], sg.at[b])
        g2 = pltpu.async_copy(d_hbm.at[dstv.at[b]], dr.at[b], sg.at[b])
        d_g.append((g1, g2))
      d_sc = []
      for b in range(NB):
        for d in d_g[b]:
          d.wait()

        def edge_body(e, _b=b):
          sv = hr[_b, e, pl.ds(d_feat, 16)]
          ex = jnp.exp(_leaky(sv + dr[_b, e]))
          hr[_b, e, pl.ds(d_feat, 16)] = ex
          for g in range(heads):
            w = ex[jnp.full((16,), g, jnp.int32)]
            for q in range(rep):
              col = (g * rep + q) * 16
              hr[_b, e, pl.ds(col, 16)] = hr[_b, e, pl.ds(col, 16)] * w

        plsc.parallel_loop(0, K, unroll=4)(edge_body)
        d_sc.append(pltpu.async_copy(hr.at[b], acc.at[dstv.at[b]], ssc.at[b],
                                     add=True))
      for d in d_sc:
        d.wait()
      return carry

    lax.fori_loop(0, NSTEP, step_body, 0)
    plsc.subcore_barrier()

    pltpu.sync_copy(acc.at[pl.ds(row0, RPW)], part_hbm.at[c, pl.ds(row0, RPW)])

  return sc_edge


_sc_edge_128 = _make_sc_edge_pass(128, HEADS)
_sc_edge_64 = _make_sc_edge_pass(64, 1)


def _attn_block(a_src, a_dst, d_feat):
  heads, dh = a_src.shape
  eye = jnp.eye(heads, dtype=jnp.float32)
  def mk(a):
    blk = a[:, :, None] * eye[:, None, :]
    m = blk.reshape(d_feat, heads)
    return jnp.pad(m, ((0, 0), (0, 16 - heads)))
  return mk(a_src), mk(a_dst)


def kernel(x, edge_index, W0, a_src0, a_dst0, b0, W1, a_src1, a_dst1, b1,
           W2, a_src2, a_dst2, b2):
  x = x.astype(jnp.float32)
  src = edge_index[0].astype(jnp.int32)
  dst = edge_index[1].astype(jnp.int32)

  As0, Ad0 = _attn_block(a_src0, a_dst0, 128)
  As1, Ad1 = _attn_block(a_src1, a_dst1, 128)
  As2, Ad2 = _attn_block(a_src2, a_dst2, 64)

  e16 = (jnp.eye(8, dtype=jnp.float32)[:, :, None]
         * jnp.ones((16,), jnp.float32)).reshape(8, 128)
  e16 = jnp.pad(e16, ((0, 8), (0, 0)))

  z144 = jnp.zeros((RPW, 144), jnp.float32)
  z80 = jnp.zeros((RPW, 80), jnp.float32)

  h0x, D0 = _tc_prep(x, W0, As0, Ad0)
  p = _sc_edge_128(h0x, D0, src, dst, z144)
  x1, h1x, D1 = _tc_fin_prep(p, h0x, D0, x, b0.reshape(1, 128), e16,
                             W1, As1, Ad1)
  p = _sc_edge_128(h1x, D1, src, dst, z144)
  x2, h2x, D2 = _tc_fin_prep(p, h1x, D1, x1, b1.reshape(1, 128), e16,
                             W2, As2, Ad2)
  p = _sc_edge_64(h2x, D2, src, dst, z80)
  out = _tc_final(p, h2x, D2, b2.reshape(1, 64))
  return out


# speedup vs baseline: 1.2174x; 1.0615x over previous
import functools

import jax
import jax.numpy as jnp
from jax import lax
from jax.experimental import pallas as pl
from jax.experimental.pallas import tpu as pltpu
from jax.experimental.pallas import tpu_sc as plsc

N = 10000
NPAD = 10240
E = 320000
HEADS = 8

NC = 2
NS = 16
NW = NC * NS
EPW = E // NW
K = 40
NCHUNK = EPW // K
NB = 5
NSTEP = NCHUNK // NB
RPW = NPAD // NS

_TCB = 2000


def _leaky(v):
  return jnp.maximum(v, v * jnp.float32(0.2))


def _prep_body(x_ref, w_ref, as_ref, ad_ref, hx_ref, d_ref):
  h = jnp.dot(x_ref[...], w_ref[...], preferred_element_type=jnp.float32)
  dout = w_ref.shape[1]
  hx_ref[:, :dout] = h
  hx_ref[:, dout:] = jnp.dot(h, as_ref[...],
                             preferred_element_type=jnp.float32)
  d_ref[...] = jnp.dot(h, ad_ref[...], preferred_element_type=jnp.float32)


def _tc_prep(x, W, As, Ad):
  dout = W.shape[1]
  grid = N // _TCB
  return pl.pallas_call(
      _prep_body,
      grid=(grid,),
      in_specs=[
          pl.BlockSpec((_TCB, x.shape[1]), lambda i: (i, 0)),
          pl.BlockSpec(W.shape, lambda i: (0, 0)),
          pl.BlockSpec(As.shape, lambda i: (0, 0)),
          pl.BlockSpec(Ad.shape, lambda i: (0, 0)),
      ],
      out_specs=[
          pl.BlockSpec((_TCB, dout + 16), lambda i: (i, 0)),
          pl.BlockSpec((_TCB, 16), lambda i: (i, 0)),
      ],
      out_shape=[
          jax.ShapeDtypeStruct((N, dout + 16), jnp.float32),
          jax.ShapeDtypeStruct((N, 16), jnp.float32),
      ],
  )(x, W, As, Ad)


def _fin_prep_body(p_ref, hx_ref, dd_ref, xres_ref, b_ref, e16_ref,
                   w_ref, as_ref, ad_ref,
                   xn_ref, hxn_ref, dn_ref):
  hx = hx_ref[...]
  h = hx[:, :128]
  s = hx[:, 128:144]
  exl = jnp.exp(_leaky(s + dd_ref[...]))
  e16 = e16_ref[...]
  exlb = jnp.dot(exl, e16, preferred_element_type=jnp.float32)
  psum = p_ref[0] + p_ref[1]
  num = psum[:, :128] + exlb * h
  den = jnp.dot(psum[:, 128:144] + exl, e16,
                preferred_element_type=jnp.float32) + jnp.float32(1e-16)
  agg = num / den + b_ref[...]
  xn = jnp.where(agg > 0, agg, jnp.exp(agg) - jnp.float32(1.0)) + xres_ref[...]
  xn_ref[...] = xn
  hn = jnp.dot(xn, w_ref[...], preferred_element_type=jnp.float32)
  dnext = w_ref.shape[1]
  hxn_ref[:, :dnext] = hn
  hxn_ref[:, dnext:] = jnp.dot(hn, as_ref[...],
                               preferred_element_type=jnp.float32)
  dn_ref[...] = jnp.dot(hn, ad_ref[...], preferred_element_type=jnp.float32)


def _tc_fin_prep(P, hx, D, xres, b, e16, W, As, Ad):
  dnext = W.shape[1]
  grid = N // _TCB
  blk = lambda shp: pl.BlockSpec(shp, lambda i: tuple(0 for _ in shp))
  row = lambda w: pl.BlockSpec((_TCB, w), lambda i: (i, 0))
  prow = lambda w: pl.BlockSpec((2, _TCB, w), lambda i: (0, i, 0))
  return pl.pallas_call(
      _fin_prep_body,
      grid=(grid,),
      in_specs=[prow(144), row(144), row(16), row(128), blk((1, 128)),
                blk((16, 128)), blk(W.shape), blk(As.shape), blk(Ad.shape)],
      out_specs=[row(128), row(dnext + 16), row(16)],
      out_shape=[
          jax.ShapeDtypeStruct((N, 128), jnp.float32),
          jax.ShapeDtypeStruct((N, dnext + 16), jnp.float32),
          jax.ShapeDtypeStruct((N, 16), jnp.float32),
      ],
  )(P, hx, D, xres, b, e16, W, As, Ad)


def _final_body(p_ref, hx_ref, dd_ref, b_ref, out_ref):
  hx = hx_ref[...]
  h2 = hx[:, :64]
  s2 = hx[:, 64:80]
  exl = jnp.exp(_leaky(s2 + dd_ref[...]))
  exl0 = exl[:, 0:1]
  psum = p_ref[0] + p_ref[1]
  den = psum[:, 64:65] + exl0 + jnp.float32(1e-16)
  o = (psum[:, :64] + exl0 * h2) / den + b_ref[...]
  m = jnp.max(o, axis=1, keepdims=True)
  om = o - m
  out_ref[...] = om - jnp.log(jnp.sum(jnp.exp(om), axis=1, keepdims=True))


def _tc_final(P, hx, D, b2):
  grid = N // _TCB
  blk = lambda shp: pl.BlockSpec(shp, lambda i: tuple(0 for _ in shp))
  row = lambda w: pl.BlockSpec((_TCB, w), lambda i: (i, 0))
  prow = lambda w: pl.BlockSpec((2, _TCB, w), lambda i: (0, i, 0))
  return pl.pallas_call(
      _final_body,
      grid=(grid,),
      in_specs=[prow(80), row(80), row(16), blk((1, 64))],
      out_specs=row(64),
      out_shape=jax.ShapeDtypeStruct((N, 64), jnp.float32),
  )(P, hx, D, b2)


def _make_sc_edge_pass(d_feat, heads):
  dext = d_feat + 16
  n_grp = d_feat // 16
  rep = n_grp // heads
  mesh = plsc.VectorSubcoreMesh(core_axis_name="c", subcore_axis_name="s",
                                num_cores=NC, num_subcores=NS)

  @functools.partial(
      pl.kernel,
      out_type=jax.ShapeDtypeStruct((NC, NPAD, dext), jnp.float32),
      mesh=mesh,
      compiler_params=pltpu.CompilerParams(use_tc_tiling_on_sc=False),
      scratch_types=[
          pltpu.VMEM((NB, K), jnp.int32),
          pltpu.VMEM((NB, K), jnp.int32),
          pltpu.VMEM((NB, K, 16), jnp.float32),
          pltpu.VMEM((NB, K, dext), jnp.float32),
          pltpu.VMEM_SHARED((NPAD, dext), jnp.float32),
          pltpu.SemaphoreType.DMA((NB,)),
          pltpu.SemaphoreType.DMA((NB,)),
          pltpu.SemaphoreType.DMA((NB,)),
      ],
  )
  def sc_edge(hx_hbm, d_hbm, src_hbm, dst_hbm, zrow_hbm, part_hbm,
              srcv, dstv, dr, hr, acc, sidx, sg, ssc):
    c = lax.axis_index("c")
    s = lax.axis_index("s")
    wid = c * NS + s
    row0 = s * RPW

    pltpu.sync_copy(zrow_hbm, acc.at[pl.ds(row0, RPW)])
    plsc.subcore_barrier()

    base = wid * EPW

    def step_body(st, carry):
      c0 = st * NB
      d_idx = []
      for b in range(NB):
        @pl.when(st > 0)
        def _(b=b):
          pltpu.make_async_copy(hr.at[b], acc.at[dstv.at[b]],
                                ssc.at[b]).wait()
        off = base + (c0 + b) * K
        d_idx.append((
            pltpu.async_copy(src_hbm.at[pl.ds(off, K)], srcv.at[b],
                             sidx.at[b]),
            pltpu.async_copy(dst_hbm.at[pl.ds(off, K)], dstv.at[b],
                             sidx.at[b])))
      d_g = []
      for b in range(NB):
        for d in d_idx[b]:
          d.wait()
        g1 = pltpu.async_copy(hx_hbm.at[srcv.at[b]], hr.at[b], sg.at[b])
        g2 = pltpu.async_copy(d_hbm.at[dstv.at[b]], dr.at[b], sg.at[b])
        d_g.append((g1, g2))
      d_sc = []
      for b in range(NB):
        for d in d_g[b]:
          d.wait()

        def edge_body(e, _b=b):
          sv = hr[_b, e, pl.ds(d_feat, 16)]
          ex = jnp.exp(_leaky(sv + dr[_b, e]))
          hr[_b, e, pl.ds(d_feat, 16)] = ex
          for g in range(heads):
            w = ex[jnp.full((16,), g, jnp.int32)]
            for q in range(rep):
              col = (g * rep + q) * 16
              hr[_b, e, pl.ds(col, 16)] = hr[_b, e, pl.ds(col, 16)] * w

        plsc.parallel_loop(0, K, unroll=4)(edge_body)
        d_sc.append(pltpu.async_copy(hr.at[b], acc.at[dstv.at[b]], ssc.at[b],
                                     add=True))
      del d_sc
      return carry

    lax.fori_loop(0, NSTEP, step_body, 0)
    for b in range(NB):
      pltpu.make_async_copy(hr.at[b], acc.at[dstv.at[b]], ssc.at[b]).wait()
    plsc.subcore_barrier()

    pltpu.sync_copy(acc.at[pl.ds(row0, RPW)], part_hbm.at[c, pl.ds(row0, RPW)])

  return sc_edge


_sc_edge_128 = _make_sc_edge_pass(128, HEADS)
_sc_edge_64 = _make_sc_edge_pass(64, 1)


def _attn_block(a_src, a_dst, d_feat):
  heads, dh = a_src.shape
  eye = jnp.eye(heads, dtype=jnp.float32)
  def mk(a):
    blk = a[:, :, None] * eye[:, None, :]
    m = blk.reshape(d_feat, heads)
    return jnp.pad(m, ((0, 0), (0, 16 - heads)))
  return mk(a_src), mk(a_dst)


def kernel(x, edge_index, W0, a_src0, a_dst0, b0, W1, a_src1, a_dst1, b1,
           W2, a_src2, a_dst2, b2):
  x = x.astype(jnp.float32)
  src = edge_index[0].astype(jnp.int32)
  dst = edge_index[1].astype(jnp.int32)

  As0, Ad0 = _attn_block(a_src0, a_dst0, 128)
  As1, Ad1 = _attn_block(a_src1, a_dst1, 128)
  As2, Ad2 = _attn_block(a_src2, a_dst2, 64)

  e16 = (jnp.eye(8, dtype=jnp.float32)[:, :, None]
         * jnp.ones((16,), jnp.float32)).reshape(8, 128)
  e16 = jnp.pad(e16, ((0, 8), (0, 0)))

  z144 = jnp.zeros((RPW, 144), jnp.float32)
  z80 = jnp.zeros((RPW, 80), jnp.float32)

  h0x, D0 = _tc_prep(x, W0, As0, Ad0)
  p = _sc_edge_128(h0x, D0, src, dst, z144)
  x1, h1x, D1 = _tc_fin_prep(p, h0x, D0, x, b0.reshape(1, 128), e16,
                             W1, As1, Ad1)
  p = _sc_edge_128(h1x, D1, src, dst, z144)
  x2, h2x, D2 = _tc_fin_prep(p, h1x, D1, x1, b1.reshape(1, 128), e16,
                             W2, As2, Ad2)
  p = _sc_edge_64(h2x, D2, src, dst, z80)
  out = _tc_final(p, h2x, D2, b2.reshape(1, 64))
  return out
